# Initial kernel scaffold; baseline (speedup 1.0000x reference)
import jax
import jax.numpy as jnp
from jax.experimental import pallas as pl


def kernel(x, table):
    raise NotImplementedError("write your pallas kernel here")


# speedup vs baseline: 1.1101x; 1.1101x over previous
import functools
import math

import jax
import jax.numpy as jnp
from jax import lax
from jax.experimental import pallas as pl
from jax.experimental.pallas import tpu as pltpu
from jax.experimental.pallas import tpu_sc as plsc

BATCH = 1024
SEQ = 200
D = 128
B = BATCH * SEQ
NC = 2
NS = 16
NW = NC * NS
B_PER_W = B // NW
CHUNK = 256
NCHUNK = B_PER_W // CHUNK
LANES = 16
SCALE = float(math.sqrt(D))


def _make_kernel():
  mesh = plsc.VectorSubcoreMesh(core_axis_name="c", subcore_axis_name="s")

  @functools.partial(
      pl.kernel,
      mesh=mesh,
      out_type=jax.ShapeDtypeStruct((B, D), jnp.float32),
      scratch_types=[
          pltpu.VMEM((CHUNK,), jnp.int32),
          pltpu.VMEM((CHUNK, D), jnp.float32),
          pltpu.SemaphoreType.DMA,
      ],
  )
  def emb_kernel(idx_hbm, table_hbm, out_hbm, idx_v, rows_v, sem):
    wid = lax.axis_index("s") * NC + lax.axis_index("c")
    base = wid * B_PER_W

    def chunk_body(g, carry):
      off = base + g * CHUNK
      pltpu.sync_copy(idx_hbm.at[pl.ds(off, CHUNK)], idx_v)
      pltpu.async_copy(table_hbm.at[idx_v], rows_v, sem).wait()

      def row_body(i, c):
        for j in range(D // LANES):
          sl = pl.ds(j * LANES, LANES)
          rows_v[i, sl] = rows_v[i, sl] * SCALE
        return c

      lax.fori_loop(0, CHUNK, row_body, 0)
      pltpu.sync_copy(rows_v, out_hbm.at[pl.ds(off, CHUNK)])
      return carry

    lax.fori_loop(0, NCHUNK, chunk_body, 0)

  return emb_kernel


_emb = _make_kernel()


def kernel(x, table):
  idx = x.reshape(-1).astype(jnp.int32)
  out = _emb(idx, table)
  return out.reshape(BATCH, SEQ, D)


# speedup vs baseline: 1.7402x; 1.5677x over previous
import functools
import math

import jax
import jax.numpy as jnp
from jax import lax
from jax.experimental import pallas as pl
from jax.experimental.pallas import tpu as pltpu
from jax.experimental.pallas import tpu_sc as plsc

BATCH = 1024
SEQ = 200
D = 128
B = BATCH * SEQ
NC = 2
NS = 16
NW = NC * NS
B_PER_W = B // NW
CHUNK = 320
NCHUNK = B_PER_W // CHUNK
NBUF = 3
LANES = 16
SCALE = float(math.sqrt(D))


def _make_kernel():
  mesh = plsc.VectorSubcoreMesh(core_axis_name="c", subcore_axis_name="s")

  @functools.partial(
      pl.kernel,
      mesh=mesh,
      out_type=jax.ShapeDtypeStruct((B, D), jnp.float32),
      scratch_types=[
          pltpu.VMEM((B_PER_W,), jnp.int32),
          pltpu.VMEM((NBUF, CHUNK, D), jnp.float32),
          pltpu.SemaphoreType.DMA((NBUF,)),
          pltpu.SemaphoreType.DMA((NBUF,)),
      ],
  )
  def emb_kernel(idx_hbm, table_hbm, out_hbm, idx_v, rows_v, gsem, wsem):
    wid = lax.axis_index("s") * NC + lax.axis_index("c")
    base = wid * B_PER_W
    pltpu.sync_copy(idx_hbm.at[pl.ds(base, B_PER_W)], idx_v)

    def start_gather(g):
      return pltpu.async_copy(
          table_hbm.at[idx_v.at[pl.ds(g * CHUNK, CHUNK)]],
          rows_v.at[g % NBUF],
          gsem.at[g % NBUF],
      )

    def scale_buf(b):
      def row_body(i, c):
        for j in range(D // LANES):
          sl = pl.ds(j * LANES, LANES)
          rows_v[b, i, sl] = rows_v[b, i, sl] * SCALE
        return c

      lax.fori_loop(0, CHUNK, row_body, 0)

    gh = [None] * NCHUNK
    wh = [None] * NCHUNK
    gh[0] = start_gather(0)
    for g in range(NCHUNK):
      if g + 1 < NCHUNK:
        if g + 1 >= NBUF:
          wh[g + 1 - NBUF].wait()
        gh[g + 1] = start_gather(g + 1)
      gh[g].wait()
      scale_buf(g % NBUF)
      wh[g] = pltpu.async_copy(
          rows_v.at[g % NBUF],
          out_hbm.at[pl.ds(base + g * CHUNK, CHUNK)],
          wsem.at[g % NBUF],
      )
    for g in range(NCHUNK - NBUF, NCHUNK):
      wh[g].wait()

  return emb_kernel


_emb = _make_kernel()


def kernel(x, table):
  idx = x.reshape(-1).astype(jnp.int32)
  out = _emb(idx, table)
  return out.reshape(BATCH, SEQ, D)
